# baseline (device time: 197062 ns/iter reference)
import jax
import jax.numpy as jnp
from jax import lax
from jax.experimental import pallas as pl
from jax.experimental.pallas import tpu as pltpu

N_DEV = 32
R_HOPS = N_DEV // 2
L_HOPS = N_DEV - 1 - R_HOPS
SUB = 4

_PLANE = [(0, 0), (1, 0), (1, 1), (0, 1), (0, 2), (1, 2), (1, 3), (0, 3)]
_COORD_OF_LOGICAL = [(x, y, z) for z in range(4) for (x, y) in _PLANE]

_H = [(0, 0), (1, 0), (2, 0), (3, 0), (3, 1), (2, 1), (1, 1), (1, 2),
      (2, 2), (3, 2), (3, 3), (2, 3), (1, 3), (0, 3), (0, 2), (0, 1)]
_RING_COORDS = [(0, y, z) for (y, z) in _H] + [(1, y, z) for (y, z) in reversed(_H)]

_LOGICAL_OF_COORD = {c: l for l, c in enumerate(_COORD_OF_LOGICAL)}
ID_AT_POS = [_LOGICAL_OF_COORD[c] for c in _RING_COORDS]
POS_OF_ID = [0] * N_DEV
for _p, _l in enumerate(ID_AT_POS):
    POS_OF_ID[_l] = _p


def kernel(x, w_mat):
    m_per, k = x.shape
    _, n_per = w_mat.shape
    sub_m = m_per // SUB

    def body(pos_tab, id_tab, x_ref, w_ref, out_ref, comm_r, comm_l,
             send_r, recv_r, send_l, recv_l):
        my = lax.axis_index("i")
        pos = pos_tab[my]

        def id_at(expr):
            return id_tab[lax.rem(expr + 2 * N_DEV, N_DEV)]

        left = id_at(pos - 1)
        right = id_at(pos + 1)

        barrier_sem = pltpu.get_barrier_semaphore()
        for nbr in (left, right):
            pl.semaphore_signal(
                barrier_sem, inc=1,
                device_id=(nbr,), device_id_type=pl.DeviceIdType.MESH,
            )
        pl.semaphore_wait(barrier_sem, 2)

        comm_r[0] = x_ref[...].astype(jnp.bfloat16)

        def sub_slice(ref, h, s):
            return ref.at[h, pl.ds(s * sub_m, sub_m), :]

        def r_rdma(h, s):
            return pltpu.make_async_remote_copy(
                src_ref=sub_slice(comm_r, h, s),
                dst_ref=sub_slice(comm_r, h + 1, s),
                send_sem=send_r.at[h, s],
                recv_sem=recv_r.at[h, s],
                device_id=(right,),
                device_id_type=pl.DeviceIdType.MESH,
            )

        def l_rdma(h, s):
            return pltpu.make_async_remote_copy(
                src_ref=sub_slice(comm_r if h == 0 else comm_l, 0 if h == 0 else h, s),
                dst_ref=sub_slice(comm_l, h + 1, s),
                send_sem=send_l.at[h, s],
                recv_sem=recv_l.at[h, s],
                device_id=(left,),
                device_id_type=pl.DeviceIdType.MESH,
            )

        w = w_ref[...].astype(jnp.bfloat16)

        def gemm_store(chunk, origin):
            y = jnp.dot(chunk, w, preferred_element_type=jnp.float32)
            y = jax.nn.gelu(y, approximate=True)
            out_ref[pl.ds(origin * m_per, m_per), :] = y

        rr = {(0, s): r_rdma(0, s) for s in range(SUB)}
        ll = {(0, s): l_rdma(0, s) for s in range(SUB)}
        for s in range(SUB):
            rr[0, s].start()
            ll[0, s].start()
        gemm_store(comm_r[0], my)

        for h in range(R_HOPS):
            has_l = h < L_HOPS
            for s in range(SUB):
                rr[h, s].wait_recv()
                if h + 1 < R_HOPS:
                    rr[h + 1, s] = r_rdma(h + 1, s)
                    rr[h + 1, s].start()
                if has_l:
                    ll[h, s].wait_recv()
                    if h + 1 < L_HOPS:
                        ll[h + 1, s] = l_rdma(h + 1, s)
                        ll[h + 1, s].start()
            gemm_store(comm_r[h + 1], id_at(pos - h - 1))
            if has_l:
                gemm_store(comm_l[h + 1], id_at(pos + h + 1))

        for rdma in list(rr.values()) + list(ll.values()):
            rdma.wait_send()

    pos_tab = jnp.asarray(POS_OF_ID, dtype=jnp.int32)
    id_tab = jnp.asarray(ID_AT_POS, dtype=jnp.int32)

    return pl.pallas_call(
        body,
        out_shape=jax.ShapeDtypeStruct((N_DEV * m_per, n_per), jnp.float32),
        in_specs=[
            pl.BlockSpec(memory_space=pltpu.SMEM),
            pl.BlockSpec(memory_space=pltpu.SMEM),
            pl.BlockSpec(memory_space=pltpu.VMEM),
            pl.BlockSpec(memory_space=pltpu.VMEM),
        ],
        out_specs=pl.BlockSpec(memory_space=pltpu.VMEM),
        scratch_shapes=[
            pltpu.VMEM((R_HOPS + 1, m_per, k), jnp.bfloat16),
            pltpu.VMEM((L_HOPS + 1, m_per, k), jnp.bfloat16),
            pltpu.SemaphoreType.DMA((R_HOPS, SUB)),
            pltpu.SemaphoreType.DMA((R_HOPS, SUB)),
            pltpu.SemaphoreType.DMA((L_HOPS, SUB)),
            pltpu.SemaphoreType.DMA((L_HOPS, SUB)),
        ],
        compiler_params=pltpu.CompilerParams(
            collective_id=0,
            vmem_limit_bytes=60 * 1024 * 1024,
        ),
    )(pos_tab, id_tab, x, w_mat)


# device time: 192684 ns/iter; 1.0227x vs baseline; 1.0227x over previous
import jax
import jax.numpy as jnp
from jax import lax
from jax.experimental import pallas as pl
from jax.experimental.pallas import tpu as pltpu

N_DEV = 32
HOPS = N_DEV // 2
SUB = 2
HALF = SUB // 2

_PLANE = [(0, 0), (1, 0), (1, 1), (0, 1), (0, 2), (1, 2), (1, 3), (0, 3)]
_COORD_OF_LOGICAL = [(x, y, z) for z in range(4) for (x, y) in _PLANE]

_H = [(0, 0), (1, 0), (2, 0), (3, 0), (3, 1), (2, 1), (1, 1), (1, 2),
      (2, 2), (3, 2), (3, 3), (2, 3), (1, 3), (0, 3), (0, 2), (0, 1)]
_RING_COORDS = [(0, y, z) for (y, z) in _H] + [(1, y, z) for (y, z) in reversed(_H)]

_LOGICAL_OF_COORD = {c: l for l, c in enumerate(_COORD_OF_LOGICAL)}
ID_AT_POS = [_LOGICAL_OF_COORD[c] for c in _RING_COORDS]
POS_OF_ID = [0] * N_DEV
for _p, _l in enumerate(ID_AT_POS):
    POS_OF_ID[_l] = _p


def _subs_r(h):
    return range(SUB) if h < HOPS - 1 else range(HALF)


def _subs_l(h):
    return range(SUB) if h < HOPS - 1 else range(HALF, SUB)


def kernel(x, w_mat):
    m_per, k = x.shape
    _, n_per = w_mat.shape
    sub_m = m_per // SUB

    def body(pos_tab, id_tab, x_ref, w_ref, out_ref, comm_r, comm_l,
             send_r, recv_r, send_l, recv_l):
        my = lax.axis_index("i")
        pos = pos_tab[my]

        def id_at(expr):
            return id_tab[lax.rem(expr + 2 * N_DEV, N_DEV)]

        left = id_at(pos - 1)
        right = id_at(pos + 1)

        barrier_sem = pltpu.get_barrier_semaphore()
        for nbr in (left, right):
            pl.semaphore_signal(
                barrier_sem, inc=1,
                device_id=(nbr,), device_id_type=pl.DeviceIdType.MESH,
            )
        pl.semaphore_wait(barrier_sem, 2)

        comm_r[0] = x_ref[...].astype(jnp.bfloat16)

        def sub_slice(ref, h, s):
            return ref.at[h, pl.ds(s * sub_m, sub_m), :]

        def r_rdma(h, s):
            return pltpu.make_async_remote_copy(
                src_ref=sub_slice(comm_r, h, s),
                dst_ref=sub_slice(comm_r, h + 1, s),
                send_sem=send_r.at[h, s],
                recv_sem=recv_r.at[h, s],
                device_id=(right,),
                device_id_type=pl.DeviceIdType.MESH,
            )

        def l_rdma(h, s):
            return pltpu.make_async_remote_copy(
                src_ref=sub_slice(comm_r if h == 0 else comm_l, 0 if h == 0 else h, s),
                dst_ref=sub_slice(comm_l, h + 1, s),
                send_sem=send_l.at[h, s],
                recv_sem=recv_l.at[h, s],
                device_id=(left,),
                device_id_type=pl.DeviceIdType.MESH,
            )

        w = w_ref[...].astype(jnp.bfloat16)

        def gemm_store(chunk, origin, row_off=0):
            y = jnp.dot(chunk, w, preferred_element_type=jnp.float32)
            y = jax.nn.gelu(y, approximate=True)
            out_ref[pl.ds(origin * m_per + row_off, chunk.shape[0]), :] = y

        rr = {(0, s): r_rdma(0, s) for s in _subs_r(0)}
        ll = {(0, s): l_rdma(0, s) for s in _subs_l(0)}
        for s in _subs_r(0):
            rr[0, s].start()
        for s in _subs_l(0):
            ll[0, s].start()
        gemm_store(comm_r[0], my)

        for h in range(HOPS):
            for s in range(SUB):
                if s in _subs_r(h):
                    rr[h, s].wait_recv()
                    if h + 1 < HOPS and s in _subs_r(h + 1):
                        rr[h + 1, s] = r_rdma(h + 1, s)
                        rr[h + 1, s].start()
                if s in _subs_l(h):
                    ll[h, s].wait_recv()
                    if h + 1 < HOPS and s in _subs_l(h + 1):
                        ll[h + 1, s] = l_rdma(h + 1, s)
                        ll[h + 1, s].start()
            if h < HOPS - 1:
                gemm_store(comm_r[h + 1], id_at(pos - h - 1))
                gemm_store(comm_l[h + 1], id_at(pos + h + 1))
            else:
                anti = id_at(pos + HOPS)
                gemm_store(comm_r[h + 1, 0:HALF * sub_m], anti, 0)
                gemm_store(comm_l[h + 1, HALF * sub_m:SUB * sub_m], anti,
                           HALF * sub_m)

        for rdma in list(rr.values()) + list(ll.values()):
            rdma.wait_send()

    pos_tab = jnp.asarray(POS_OF_ID, dtype=jnp.int32)
    id_tab = jnp.asarray(ID_AT_POS, dtype=jnp.int32)

    return pl.pallas_call(
        body,
        out_shape=jax.ShapeDtypeStruct((N_DEV * m_per, n_per), jnp.float32),
        in_specs=[
            pl.BlockSpec(memory_space=pltpu.SMEM),
            pl.BlockSpec(memory_space=pltpu.SMEM),
            pl.BlockSpec(memory_space=pltpu.VMEM),
            pl.BlockSpec(memory_space=pltpu.VMEM),
        ],
        out_specs=pl.BlockSpec(memory_space=pltpu.VMEM),
        scratch_shapes=[
            pltpu.VMEM((HOPS + 1, m_per, k), jnp.bfloat16),
            pltpu.VMEM((HOPS + 1, m_per, k), jnp.bfloat16),
            pltpu.SemaphoreType.DMA((HOPS, SUB)),
            pltpu.SemaphoreType.DMA((HOPS, SUB)),
            pltpu.SemaphoreType.DMA((HOPS, SUB)),
            pltpu.SemaphoreType.DMA((HOPS, SUB)),
        ],
        compiler_params=pltpu.CompilerParams(
            collective_id=0,
            vmem_limit_bytes=60 * 1024 * 1024,
        ),
    )(pos_tab, id_tab, x, w_mat)
